# Initial kernel scaffold; baseline (speedup 1.0000x reference)
import jax
import jax.numpy as jnp
from jax.experimental import pallas as pl


def kernel(images, augmented_images):
    raise NotImplementedError("write your pallas kernel here")


# speedup vs baseline: 1.7582x; 1.7582x over previous
import jax
import jax.numpy as jnp
from jax.experimental import pallas as pl
from jax.experimental.pallas import tpu as pltpu

_THRESHOLD = 0.012
_B, _C = 16384, 1000
_BLK = 1024
_NBLK = _B // _BLK


def _ce_loss_kernel(img_ref, aug_ref, out_ref, counts_ref, snll_ref):
    i = pl.program_id(0)

    @pl.when(i == 0)
    def _init():
        counts_ref[...] = jnp.zeros_like(counts_ref)
        snll_ref[...] = jnp.zeros_like(snll_ref)

    img = img_ref[...]
    aug = aug_ref[...]

    m = jnp.max(img, axis=1, keepdims=True)
    s = jnp.sum(jnp.exp(img - m), axis=1)
    cols = jax.lax.broadcasted_iota(jnp.int32, (_BLK, _C), 1)
    lbl = jnp.min(jnp.where(img == m, cols, _C), axis=1)
    maxprob = 1.0 / s
    mask = (maxprob > _THRESHOLD).astype(jnp.float32)

    am = jnp.max(aug, axis=1, keepdims=True)
    alse = am[:, 0] + jnp.log(jnp.sum(jnp.exp(aug - am), axis=1))
    onehot = (cols == lbl[:, None]).astype(jnp.float32)
    taken = jnp.sum(onehot * aug, axis=1)
    nll = alse - taken

    oh_m = onehot * mask[:, None]
    counts_ref[...] += jnp.sum(oh_m, axis=0)[None, :]
    snll_ref[...] += jnp.sum(oh_m * nll[:, None], axis=0)[None, :]

    @pl.when(i == _NBLK - 1)
    def _finish():
        counts = counts_ref[0, :]
        snll = snll_ref[0, :]
        present = counts > 0
        k = jnp.sum(present.astype(jnp.float32))
        per_class = jnp.where(present, snll / jnp.where(present, counts, 1.0), 0.0)
        out_ref[...] = (jnp.sum(per_class) / k).reshape(1, 1)


def kernel(images, augmented_images):
    out = pl.pallas_call(
        _ce_loss_kernel,
        grid=(_NBLK,),
        in_specs=[
            pl.BlockSpec((_BLK, _C), lambda i: (i, 0)),
            pl.BlockSpec((_BLK, _C), lambda i: (i, 0)),
        ],
        out_specs=pl.BlockSpec((1, 1), lambda i: (0, 0)),
        out_shape=jax.ShapeDtypeStruct((1, 1), jnp.float32),
        scratch_shapes=[
            pltpu.VMEM((1, _C), jnp.float32),
            pltpu.VMEM((1, _C), jnp.float32),
        ],
    )(images, augmented_images)
    return out[0, 0]


# speedup vs baseline: 1.7865x; 1.0161x over previous
import jax
import jax.numpy as jnp
from jax.experimental import pallas as pl
from jax.experimental.pallas import tpu as pltpu

_THRESHOLD = 0.012
_B, _C = 16384, 1000
_BLK = 1024
_NBLK = _B // _BLK


def _ce_loss_kernel(img_ref, aug_ref, out_ref, seg_ref):
    i = pl.program_id(0)

    @pl.when(i == 0)
    def _init():
        seg_ref[...] = jnp.zeros_like(seg_ref)

    img = img_ref[...]
    aug = aug_ref[...]

    m = jnp.max(img, axis=1, keepdims=True)
    s = jnp.sum(jnp.exp(img - m), axis=1)
    cols = jax.lax.broadcasted_iota(jnp.int32, (_BLK, _C), 1)
    lbl = jnp.min(jnp.where(img == m, cols, _C), axis=1)
    maxprob = 1.0 / s
    mask = (maxprob > _THRESHOLD).astype(jnp.float32)

    am = jnp.max(aug, axis=1, keepdims=True)
    alse = am[:, 0] + jnp.log(jnp.sum(jnp.exp(aug - am), axis=1))
    onehot = (cols == lbl[:, None]).astype(jnp.float32)
    taken = jnp.sum(onehot * aug, axis=1)
    nll = alse - taken

    lhs = jnp.stack([mask, mask * nll], axis=0)
    acc = jax.lax.dot_general(
        lhs, onehot, (((1,), (0,)), ((), ())),
        preferred_element_type=jnp.float32)
    seg_ref[...] += acc

    @pl.when(i == _NBLK - 1)
    def _finish():
        counts = seg_ref[0, :]
        snll = seg_ref[1, :]
        present = counts > 0
        k = jnp.sum(present.astype(jnp.float32))
        per_class = jnp.where(present, snll / jnp.where(present, counts, 1.0), 0.0)
        out_ref[...] = (jnp.sum(per_class) / k).reshape(1, 1)


def kernel(images, augmented_images):
    out = pl.pallas_call(
        _ce_loss_kernel,
        grid=(_NBLK,),
        in_specs=[
            pl.BlockSpec((_BLK, _C), lambda i: (i, 0)),
            pl.BlockSpec((_BLK, _C), lambda i: (i, 0)),
        ],
        out_specs=pl.BlockSpec((1, 1), lambda i: (0, 0)),
        out_shape=jax.ShapeDtypeStruct((1, 1), jnp.float32),
        scratch_shapes=[
            pltpu.VMEM((2, _C), jnp.float32),
        ],
    )(images, augmented_images)
    return out[0, 0]
